# Initial kernel scaffold; baseline (speedup 1.0000x reference)
import jax
import jax.numpy as jnp
from jax.experimental import pallas as pl


def kernel(edge_rel_pos, table):
    raise NotImplementedError("write your pallas kernel here")


# speedup vs baseline: 2.4217x; 2.4217x over previous
import jax
import jax.numpy as jnp
from jax import lax
from jax.experimental import pallas as pl
from jax.experimental.pallas import tpu as pltpu
from jax.experimental.pallas import tpu_sc as plsc

_HEADS = 16
_NC, _NS = 2, 16
_NW = _NC * _NS
_SUB = 128
_ROWS_PER_CHUNK = 16
_CHUNK = _SUB * _ROWS_PER_CHUNK


def _make_lookup(n_idx):
  assert n_idx % (_NW * _CHUNK) == 0
  per_w = n_idx // _NW
  n_chunks = per_w // _CHUNK
  idx_rows_per_w = per_w // _SUB

  def body(table_hbm, idx_hbm, out_hbm, idx_v, rows_v, sem_g):
    c = lax.axis_index("c")
    s = lax.axis_index("s")
    w = s * _NC + c
    idx_row0 = w * idx_rows_per_w
    out_row0 = w * per_w

    @pl.loop(0, n_chunks)
    def _chunk(g):
      pltpu.sync_copy(
          idx_hbm.at[pl.ds(idx_row0 + g * _ROWS_PER_CHUNK, _ROWS_PER_CHUNK)],
          idx_v)
      for j in range(_ROWS_PER_CHUNK):
        pltpu.async_copy(table_hbm.at[idx_v.at[j]],
                         rows_v.at[pl.ds(j * _SUB, _SUB)], sem_g)
      for j in range(_ROWS_PER_CHUNK):
        pltpu.make_async_copy(table_hbm.at[idx_v.at[j]],
                              rows_v.at[pl.ds(j * _SUB, _SUB)], sem_g).wait()
      pltpu.sync_copy(rows_v, out_hbm.at[pl.ds(out_row0 + g * _CHUNK, _CHUNK)])

  return pl.kernel(
      body,
      out_type=jax.ShapeDtypeStruct((n_idx, _HEADS), jnp.float32),
      mesh=plsc.VectorSubcoreMesh(core_axis_name="c", subcore_axis_name="s",
                                  num_cores=_NC, num_subcores=_NS),
      scratch_types=[
          pltpu.VMEM((_ROWS_PER_CHUNK, _SUB), jnp.int32),
          pltpu.VMEM((_CHUNK, _HEADS), jnp.float32),
          pltpu.SemaphoreType.DMA,
      ],
      compiler_params=pltpu.CompilerParams(use_tc_tiling_on_sc=False),
  )


def kernel(edge_rel_pos, table):
  shape = edge_rel_pos.shape
  n_idx = edge_rel_pos.size
  idx = edge_rel_pos.reshape(n_idx // _SUB, _SUB).astype(jnp.int32)
  out = _make_lookup(n_idx)(table.astype(jnp.float32), idx)
  return out.reshape(shape + (_HEADS,))


# speedup vs baseline: 7.5748x; 3.1279x over previous
import jax
import jax.numpy as jnp
from jax import lax
from jax.experimental import pallas as pl
from jax.experimental.pallas import tpu as pltpu
from jax.experimental.pallas import tpu_sc as plsc

_HEADS = 16
_VOCAB = 32
_NC, _NS = 2, 16
_NW = _NC * _NS
_CHUNK = 2048
_UNROLL = 16


def _make_lookup(n_idx):
  assert n_idx % (_NW * _CHUNK) == 0
  per_w = n_idx // _NW
  n_chunks = per_w // _CHUNK
  assert n_chunks % 2 == 0

  def body(table_hbm, idx_hbm, out_hbm, table_v, idx_v, rows_v,
           sem_i0, sem_i1, sem_o0, sem_o1):
    c = lax.axis_index("c")
    s = lax.axis_index("s")
    w = s * _NC + c
    idx0 = w * per_w

    sems_i = (sem_i0, sem_i1)
    sems_o = (sem_o0, sem_o1)

    def in_copy(g, b):
      return pltpu.make_async_copy(
          idx_hbm.at[pl.ds(idx0 + g * _CHUNK, _CHUNK)], idx_v.at[b],
          sems_i[b])

    def out_copy(g, b):
      return pltpu.make_async_copy(
          rows_v.at[b], out_hbm.at[pl.ds(idx0 + g * _CHUNK, _CHUNK)],
          sems_o[b])

    pltpu.sync_copy(table_hbm, table_v)
    in_copy(0, 0).start()

    @pl.loop(0, n_chunks, step=2)
    def _pair(g0):
      for b in range(2):
        g = g0 + b

        @pl.when(g + 1 < n_chunks)
        def _prefetch():
          in_copy(g + 1, 1 - b).start()

        in_copy(g, b).wait()

        @pl.when(g >= 2)
        def _drain():
          out_copy(g - 2, b).wait()

        idx_vb = idx_v.at[b]
        rows_vb = rows_v.at[b]

        @pl.loop(0, _CHUNK // _UNROLL)
        def _grp(j):
          base = j * _UNROLL
          iv = idx_vb[pl.ds(base, _UNROLL)]
          for u in range(_UNROLL):
            rows_vb[base + u] = table_v[iv[u]]

        out_copy(g, b).start()

    out_copy(n_chunks - 2, 0).wait()
    out_copy(n_chunks - 1, 1).wait()

  return pl.kernel(
      body,
      out_type=jax.ShapeDtypeStruct((n_idx, _HEADS), jnp.float32),
      mesh=plsc.VectorSubcoreMesh(core_axis_name="c", subcore_axis_name="s",
                                  num_cores=_NC, num_subcores=_NS),
      scratch_types=[
          pltpu.VMEM((_VOCAB, _HEADS), jnp.float32),
          pltpu.VMEM((2, _CHUNK), jnp.int32),
          pltpu.VMEM((2, _CHUNK, _HEADS), jnp.float32),
          pltpu.SemaphoreType.DMA,
          pltpu.SemaphoreType.DMA,
          pltpu.SemaphoreType.DMA,
          pltpu.SemaphoreType.DMA,
      ],
      compiler_params=pltpu.CompilerParams(use_tc_tiling_on_sc=False),
  )


def kernel(edge_rel_pos, table):
  shape = edge_rel_pos.shape
  n_idx = edge_rel_pos.size
  idx = edge_rel_pos.reshape(n_idx).astype(jnp.int32)
  out = _make_lookup(n_idx)(table.astype(jnp.float32), idx)
  return out.reshape(shape + (_HEADS,))
